# baseline (speedup 1.0000x reference)
import functools

import jax
import jax.numpy as jnp
from jax import lax
from jax.experimental import pallas as pl
from jax.experimental.pallas import tpu as pltpu
from jax.experimental.pallas import tpu_sc as plsc


_CHUNK = 80


def _sc_gather(table, idx_flat):
    V, E = table.shape
    N = idx_flat.shape[0]
    info = plsc.get_sparse_core_info()
    NC, NS = info.num_cores, info.num_subcores
    NW = NC * NS
    assert N % (NW * _CHUNK) == 0
    n_per_w = N // NW
    n_chunks = n_per_w // _CHUNK
    idx3 = idx_flat.reshape(NW, n_chunks, _CHUNK)

    mesh = plsc.VectorSubcoreMesh(core_axis_name="c", subcore_axis_name="s")

    @functools.partial(
        pl.kernel,
        mesh=mesh,
        out_type=jax.ShapeDtypeStruct((N, E), jnp.float32),
        scratch_types=[
            pltpu.VMEM((n_chunks, _CHUNK), jnp.int32),
            pltpu.VMEM((n_per_w, E), jnp.float32),
            pltpu.SemaphoreType.DMA,
        ],
        compiler_params=pltpu.CompilerParams(use_tc_tiling_on_sc=False),
    )
    def gather_kernel(table_hbm, idx_hbm, out_hbm, idx_v, rows_v, sem):
        wid = lax.axis_index("s") * NC + lax.axis_index("c")
        pltpu.sync_copy(idx_hbm.at[wid], idx_v)
        copies = [
            pltpu.async_copy(
                table_hbm.at[idx_v.at[j]],
                rows_v.at[pl.ds(j * _CHUNK, _CHUNK)],
                sem,
            )
            for j in range(n_chunks)
        ]
        for c in copies:
            c.wait()
        pltpu.sync_copy(rows_v, out_hbm.at[pl.ds(wid * n_per_w, n_per_w)])

    return gather_kernel(table, idx3)


def _rnn_step(emb_ref, pm_ref, lens_ref, wih_ref, whh_ref, b_ref,
              out_ref, hid_ref, h_ref, *, L, H):
    t = pl.program_id(0)

    @pl.when(t == 0)
    def _init():
        h_ref[...] = jnp.zeros_like(h_ref)

    x = emb_ref[0] * pm_ref[0]
    h = h_ref[...]
    acc = jnp.dot(x, wih_ref[...], preferred_element_type=jnp.float32)
    acc = acc + jnp.dot(h, whh_ref[...], preferred_element_type=jnp.float32)
    h_new = jnp.tanh(acc + b_ref[...])
    valid = t < lens_ref[...]
    h_next = jnp.where(valid, h_new, h)
    h_ref[...] = h_next
    lane = pl.multiple_of(t * H, H)
    out_ref[:, pl.ds(lane, H)] = jnp.where(valid, h_new, 0.0)

    @pl.when(t == L - 1)
    def _fin():
        hid_ref[...] = h_next


def _tc_rnn(emb, pmask, lens2, wih_t, whh_t, bias, *, interpret=False):
    L, B, E = emb.shape
    H = whh_t.shape[0]
    grid = (L,)
    out_shapes = (
        jax.ShapeDtypeStruct((B, L * H), jnp.float32),
        jax.ShapeDtypeStruct((B, H), jnp.float32),
    )
    return pl.pallas_call(
        functools.partial(_rnn_step, L=L, H=H),
        grid=grid,
        in_specs=[
            pl.BlockSpec((1, B, E), lambda t: (t, 0, 0)),
            pl.BlockSpec((1, B, 1), lambda t: (t, 0, 0)),
            pl.BlockSpec((B, 1), lambda t: (0, 0)),
            pl.BlockSpec((E, H), lambda t: (0, 0)),
            pl.BlockSpec((H, H), lambda t: (0, 0)),
            pl.BlockSpec((1, H), lambda t: (0, 0)),
        ],
        out_specs=(
            pl.BlockSpec((B, L * H), lambda t: (0, 0)),
            pl.BlockSpec((B, H), lambda t: (0, 0)),
        ),
        out_shape=out_shapes,
        scratch_shapes=[pltpu.VMEM((B, H), jnp.float32)],
        compiler_params=pltpu.CompilerParams(
            dimension_semantics=("arbitrary",),
        ),
        interpret=interpret,
    )(emb, pmask, lens2, wih_t, whh_t, bias)


def kernel(src, lens, table, W_ih, W_hh, b_ih, b_hh):
    B, L = src.shape
    V, E = table.shape
    H = W_hh.shape[0]

    idx_flat = src.T.reshape(-1)
    emb_flat = _sc_gather(table, idx_flat)
    emb = emb_flat.reshape(L, B, E)

    pmask = (idx_flat != 0).astype(jnp.float32).reshape(L, B, 1)
    lens2 = lens.astype(jnp.int32).reshape(B, 1)
    bias = (b_ih + b_hh).reshape(1, H)

    out, hT = _tc_rnn(emb, pmask, lens2, W_ih.T, W_hh.T, bias)
    return out.reshape(B, L, H), hT[None]


# speedup vs baseline: 1.0150x; 1.0150x over previous
import functools

import jax
import jax.numpy as jnp
from jax import lax
from jax.experimental import pallas as pl
from jax.experimental.pallas import tpu as pltpu
from jax.experimental.pallas import tpu_sc as plsc

_CHUNK = 80


def _sc_gather(table2, idx_flat):
    V2, D = table2.shape
    N = idx_flat.shape[0]
    info = plsc.get_sparse_core_info()
    NC, NS, LN = info.num_cores, info.num_subcores, info.num_lanes
    NW = NC * NS
    assert N % (NW * _CHUNK) == 0
    n_per_w = N // NW
    n_chunks = n_per_w // _CHUNK
    half_chunks = n_chunks // 2
    half_rows = n_per_w // 2
    idx3 = idx_flat.reshape(NW, n_chunks, _CHUNK)

    mesh = plsc.VectorSubcoreMesh(core_axis_name="c", subcore_axis_name="s")

    @functools.partial(
        pl.kernel,
        mesh=mesh,
        out_type=(
            jax.ShapeDtypeStruct((N, D), jnp.float32),
            jax.ShapeDtypeStruct((N,), jnp.float32),
            jax.ShapeDtypeStruct((N,), jnp.float32),
        ),
        scratch_types=[
            pltpu.VMEM((n_chunks, _CHUNK), jnp.int32),
            pltpu.VMEM((n_chunks, _CHUNK), jnp.int32),
            pltpu.VMEM((half_rows, D), jnp.float32),
            pltpu.VMEM((n_per_w,), jnp.float32),
            pltpu.VMEM((n_per_w,), jnp.float32),
            pltpu.SemaphoreType.DMA,
        ],
    )
    def gather_kernel(table_hbm, idx_hbm, out_hbm, enc0_hbm, enc1_hbm,
                      idx_v, idxg_v, rows_v, enc0_v, enc1_v, sem):
        wid = lax.axis_index("s") * NC + lax.axis_index("c")
        base = wid * n_per_w
        pltpu.sync_copy(idx_hbm.at[wid], idx_v)
        for j in range(n_chunks):
            for k in range(_CHUNK // LN):
                sl = pl.ds(k * LN, LN)
                v = idx_v[j, sl]
                idxg_v[j, sl] = lax.shift_right_logical(v, 1)
                e1 = (v & 1).astype(jnp.float32)
                nz = jnp.minimum(v, 1).astype(jnp.float32)
                fsl = pl.ds(j * _CHUNK + k * LN, LN)
                enc0_v[fsl] = nz - e1
                enc1_v[fsl] = e1
        for half in range(2):
            copies = [
                pltpu.async_copy(
                    table_hbm.at[idxg_v.at[half * half_chunks + j]],
                    rows_v.at[pl.ds(j * _CHUNK, _CHUNK)],
                    sem,
                )
                for j in range(half_chunks)
            ]
            for c in copies:
                c.wait()
            pltpu.sync_copy(
                rows_v, out_hbm.at[pl.ds(base + half * half_rows, half_rows)]
            )
        pltpu.sync_copy(enc0_v, enc0_hbm.at[pl.ds(base, n_per_w)])
        pltpu.sync_copy(enc1_v, enc1_hbm.at[pl.ds(base, n_per_w)])

    return gather_kernel(table2, idx3)


def _rnn_step(emb_ref, e0_ref, e1_ref, lens_ref, w2_ref, whh_ref, b_ref,
              out_ref, hid_ref, h_ref, *, L, H):
    t = pl.program_id(0)

    @pl.when(t == 0)
    def _init():
        h_ref[...] = jnp.zeros_like(h_ref)

    p = jnp.dot(emb_ref[0], w2_ref[...], preferred_element_type=jnp.float32)
    x = p[:, :H] * e0_ref[0] + p[:, H:] * e1_ref[0]
    h = h_ref[...]
    acc = x + jnp.dot(h, whh_ref[...], preferred_element_type=jnp.float32)
    h_new = jnp.tanh(acc + b_ref[...])
    valid = t < lens_ref[...]
    h_next = jnp.where(valid, h_new, h)
    h_ref[...] = h_next
    out_ref[0] = jnp.where(valid, h_new, 0.0)

    @pl.when(t == L - 1)
    def _fin():
        hid_ref[...] = h_next


def _tc_rnn(emb2, enc0, enc1, lens2, w2, whh_t, bias, *, interpret=False):
    L, B, D = emb2.shape
    H = whh_t.shape[0]
    grid = (L,)
    out_shapes = (
        jax.ShapeDtypeStruct((L, B, H), jnp.float32),
        jax.ShapeDtypeStruct((B, H), jnp.float32),
    )
    return pl.pallas_call(
        functools.partial(_rnn_step, L=L, H=H),
        grid=grid,
        in_specs=[
            pl.BlockSpec((1, B, D), lambda t: (t, 0, 0)),
            pl.BlockSpec((1, B, 1), lambda t: (t, 0, 0)),
            pl.BlockSpec((1, B, 1), lambda t: (t, 0, 0)),
            pl.BlockSpec((B, 1), lambda t: (0, 0)),
            pl.BlockSpec((D, 2 * H), lambda t: (0, 0)),
            pl.BlockSpec((H, H), lambda t: (0, 0)),
            pl.BlockSpec((1, H), lambda t: (0, 0)),
        ],
        out_specs=(
            pl.BlockSpec((1, B, H), lambda t: (t, 0, 0)),
            pl.BlockSpec((B, H), lambda t: (0, 0)),
        ),
        out_shape=out_shapes,
        scratch_shapes=[pltpu.VMEM((B, H), jnp.float32)],
        compiler_params=pltpu.CompilerParams(
            dimension_semantics=("arbitrary",),
        ),
        interpret=interpret,
    )(emb2, enc0, enc1, lens2, w2, whh_t, bias)


def kernel(src, lens, table, W_ih, W_hh, b_ih, b_hh):
    B, L = src.shape
    V, E = table.shape
    H = W_hh.shape[0]

    table2 = table.reshape(V // 2, 2 * E)
    idx_flat = src.T.reshape(-1)
    rows, enc0, enc1 = _sc_gather(table2, idx_flat)
    emb2 = rows.reshape(L, B, 2 * E)
    enc0 = enc0.reshape(L, B, 1)
    enc1 = enc1.reshape(L, B, 1)

    lens2 = lens.astype(jnp.int32).reshape(B, 1)
    wih_t = W_ih.T
    w2 = jnp.zeros((2 * E, 2 * H), jnp.float32)
    w2 = w2.at[:E, :H].set(wih_t).at[E:, H:].set(wih_t)
    bias = (b_ih + b_hh).reshape(1, H)

    out, hT = _tc_rnn(emb2, enc0, enc1, lens2, w2, W_hh.T, bias)
    return jnp.transpose(out, (1, 0, 2)), hT[None]


# speedup vs baseline: 1.4101x; 1.3893x over previous
import functools

import jax
import jax.numpy as jnp
from jax import lax
from jax.experimental import pallas as pl
from jax.experimental.pallas import tpu as pltpu
from jax.experimental.pallas import tpu_sc as plsc

_CHUNK = 80


def _sc_gather(table2, idx3):
    V2, D = table2.shape
    NW, n_chunks, C = idx3.shape
    N = NW * n_chunks * C
    n_per_w = n_chunks * C
    half_chunks = n_chunks // 2
    half_rows = n_per_w // 2

    mesh = plsc.VectorSubcoreMesh(core_axis_name="c", subcore_axis_name="s")

    @functools.partial(
        pl.kernel,
        mesh=mesh,
        out_type=jax.ShapeDtypeStruct((N, D), jnp.float32),
        scratch_types=[
            pltpu.VMEM((n_chunks, C), jnp.int32),
            pltpu.VMEM((half_rows, D), jnp.float32),
            pltpu.SemaphoreType.DMA,
        ],
    )
    def gather_kernel(table_hbm, idx_hbm, out_hbm, idx_v, rows_v, sem):
        nc = lax.axis_index("c")
        ns = lax.axis_index("s")
        wid = ns * 2 + nc
        base = wid * n_per_w
        pltpu.sync_copy(idx_hbm.at[wid], idx_v)
        for half in range(2):
            copies = [
                pltpu.async_copy(
                    table_hbm.at[idx_v.at[half * half_chunks + j]],
                    rows_v.at[pl.ds(j * C, C)],
                    sem,
                )
                for j in range(half_chunks)
            ]
            for c in copies:
                c.wait()
            pltpu.sync_copy(
                rows_v, out_hbm.at[pl.ds(base + half * half_rows, half_rows)]
            )

    return gather_kernel(table2, idx3)


def _rnn_step(emb_ref, e01_ref, lens_ref, w2_ref, whh_ref, b_ref,
              out_ref, hid_ref, h_ref, *, L, H):
    t = pl.program_id(0)

    @pl.when(t == 0)
    def _init():
        h_ref[...] = jnp.zeros_like(h_ref)

    p = jnp.dot(emb_ref[0], w2_ref[...], preferred_element_type=jnp.float32)
    r = lax.broadcasted_iota(jnp.int32, (2 * L, 2), 0)
    c = lax.broadcasted_iota(jnp.int32, (2 * L, 2), 1)
    sel = (r == t + c * L).astype(jnp.float32)
    e = jnp.dot(e01_ref[...], sel, preferred_element_type=jnp.float32)
    e0 = e[:, 0:1]
    e1 = e[:, 1:2]
    x = p[:, :H] * e0 + p[:, H:] * e1
    h = h_ref[...]
    acc = x + jnp.dot(h, whh_ref[...], preferred_element_type=jnp.float32)
    h_new = jnp.tanh(acc + b_ref[...])
    valid = t < lens_ref[...]
    h_next = jnp.where(valid, h_new, h)
    h_ref[...] = h_next
    out_ref[0] = jnp.where(valid, h_new, 0.0)

    @pl.when(t == L - 1)
    def _fin():
        hid_ref[...] = h_next


def _tc_rnn(emb2, e01, lens2, w2, whh_t, bias, *, interpret=False):
    L, B, D = emb2.shape
    H = whh_t.shape[0]
    grid = (L,)
    out_shapes = (
        jax.ShapeDtypeStruct((L, B, H), jnp.float32),
        jax.ShapeDtypeStruct((B, H), jnp.float32),
    )
    return pl.pallas_call(
        functools.partial(_rnn_step, L=L, H=H),
        grid=grid,
        in_specs=[
            pl.BlockSpec((1, B, D), lambda t: (t, 0, 0)),
            pl.BlockSpec((B, 2 * L), lambda t: (0, 0)),
            pl.BlockSpec((B, 1), lambda t: (0, 0)),
            pl.BlockSpec((D, 2 * H), lambda t: (0, 0)),
            pl.BlockSpec((H, H), lambda t: (0, 0)),
            pl.BlockSpec((1, H), lambda t: (0, 0)),
        ],
        out_specs=(
            pl.BlockSpec((1, B, H), lambda t: (t, 0, 0)),
            pl.BlockSpec((B, H), lambda t: (0, 0)),
        ),
        out_shape=out_shapes,
        scratch_shapes=[pltpu.VMEM((B, H), jnp.float32)],
        compiler_params=pltpu.CompilerParams(
            dimension_semantics=("arbitrary",),
        ),
        interpret=interpret,
    )(emb2, e01, lens2, w2, whh_t, bias)


def kernel(src, lens, table, W_ih, W_hh, b_ih, b_hh):
    B, L = src.shape
    V, E = table.shape
    H = W_hh.shape[0]
    NW = 32

    table2 = table.reshape(V // 2, 2 * E)
    idx_half = (src.T.reshape(-1) >> 1).astype(jnp.int32)
    n_chunks = (L * B) // (NW * _CHUNK)
    idx3 = idx_half.reshape(NW, n_chunks, _CHUNK)
    rows = _sc_gather(table2, idx3)
    emb2 = rows.reshape(L, B, 2 * E)

    lens2 = lens.astype(jnp.int32).reshape(B, 1)
    wih_t = W_ih.T
    w2 = jnp.zeros((2 * E, 2 * H), jnp.float32)
    w2 = w2.at[:E, :H].set(wih_t).at[E:, H:].set(wih_t)
    bias = (b_ih + b_hh).reshape(1, H)

    odd = (src % 2 == 1)
    e0m = ((src != 0) & ~odd).astype(jnp.float32)
    e1m = odd.astype(jnp.float32)
    e01 = jnp.concatenate([e0m, e1m], axis=1)

    out, hT = _tc_rnn(emb2, e01, lens2, w2, W_hh.T, bias)
    return jnp.transpose(out, (1, 0, 2)), hT[None]


# speedup vs baseline: 1.5707x; 1.1139x over previous
import functools

import jax
import jax.numpy as jnp
from jax import lax
from jax.experimental import pallas as pl
from jax.experimental.pallas import tpu as pltpu
from jax.experimental.pallas import tpu_sc as plsc

_CHUNK = 80


def _sc_gather(table2, idx3):
    V2, D = table2.shape
    NW, n_chunks, C = idx3.shape
    N = NW * n_chunks * C
    n_per_w = n_chunks * C
    half_chunks = n_chunks // 2
    half_rows = n_per_w // 2

    mesh = plsc.VectorSubcoreMesh(core_axis_name="c", subcore_axis_name="s")

    @functools.partial(
        pl.kernel,
        mesh=mesh,
        out_type=jax.ShapeDtypeStruct((N, D), jnp.float32),
        scratch_types=[
            pltpu.VMEM((n_chunks, C), jnp.int32),
            pltpu.VMEM((half_rows, D), jnp.float32),
            pltpu.SemaphoreType.DMA,
        ],
    )
    def gather_kernel(table_hbm, idx_hbm, out_hbm, idx_v, rows_v, sem):
        nc = lax.axis_index("c")
        ns = lax.axis_index("s")
        wid = ns * 2 + nc
        base = wid * n_per_w
        pltpu.sync_copy(idx_hbm.at[wid], idx_v)
        for half in range(2):
            copies = [
                pltpu.async_copy(
                    table_hbm.at[idx_v.at[half * half_chunks + j]],
                    rows_v.at[pl.ds(j * C, C)],
                    sem,
                )
                for j in range(half_chunks)
            ]
            for c in copies:
                c.wait()
            pltpu.sync_copy(
                rows_v, out_hbm.at[pl.ds(base + half * half_rows, half_rows)]
            )

    return gather_kernel(table2, idx3)


def _transpose_step(lo_ref, hi_ref, out_ref, *, E):
    out_ref[:, :E] = jnp.transpose(lo_ref[...], (1, 0))
    out_ref[:, E:] = jnp.transpose(hi_ref[...], (1, 0))


def _tc_table(tableT, n_blocks, ct):
    E, V = tableT.shape
    half = n_blocks * ct
    return pl.pallas_call(
        functools.partial(_transpose_step, E=E),
        grid=(n_blocks,),
        in_specs=[
            pl.BlockSpec((E, ct), lambda g: (0, g)),
            pl.BlockSpec((E, ct), lambda g, nb=n_blocks: (0, g + nb)),
        ],
        out_specs=pl.BlockSpec((ct, 2 * E), lambda g: (g, 0)),
        out_shape=jax.ShapeDtypeStruct((half, 2 * E), jnp.float32),
    )(tableT, tableT)


def _rnn_step(emb_ref, e01_ref, lens_ref, w2_ref, whh_ref, b_ref,
              out_ref, hid_ref, h_ref, *, L, H):
    t = pl.program_id(0)

    @pl.when(t == 0)
    def _init():
        h_ref[...] = jnp.zeros_like(h_ref)

    p = jnp.dot(emb_ref[0], w2_ref[...], preferred_element_type=jnp.float32)
    r = lax.broadcasted_iota(jnp.int32, (2 * L, 2), 0)
    c = lax.broadcasted_iota(jnp.int32, (2 * L, 2), 1)
    sel = (r == t + c * L).astype(jnp.float32)
    e = jnp.dot(e01_ref[...], sel, preferred_element_type=jnp.float32)
    e0 = e[:, 0:1]
    e1 = e[:, 1:2]
    x = p[:, :H] * e0 + p[:, H:] * e1
    h = h_ref[...]
    acc = x + jnp.dot(h, whh_ref[...], preferred_element_type=jnp.float32)
    h_new = jnp.tanh(acc + b_ref[...])
    valid = t < lens_ref[...]
    h_next = jnp.where(valid, h_new, h)
    h_ref[...] = h_next
    out_ref[0] = jnp.where(valid, h_new, 0.0)

    @pl.when(t == L - 1)
    def _fin():
        hid_ref[...] = h_next


def _tc_rnn(emb2, e01, lens2, w2, whh_t, bias, *, interpret=False):
    L, B, D = emb2.shape
    H = whh_t.shape[0]
    grid = (L,)
    out_shapes = (
        jax.ShapeDtypeStruct((L, B, H), jnp.float32),
        jax.ShapeDtypeStruct((B, H), jnp.float32),
    )
    return pl.pallas_call(
        functools.partial(_rnn_step, L=L, H=H),
        grid=grid,
        in_specs=[
            pl.BlockSpec((1, B, D), lambda t: (t, 0, 0)),
            pl.BlockSpec((B, 2 * L), lambda t: (0, 0)),
            pl.BlockSpec((B, 1), lambda t: (0, 0)),
            pl.BlockSpec((D, 2 * H), lambda t: (0, 0)),
            pl.BlockSpec((H, H), lambda t: (0, 0)),
            pl.BlockSpec((1, H), lambda t: (0, 0)),
        ],
        out_specs=(
            pl.BlockSpec((1, B, H), lambda t: (t, 0, 0)),
            pl.BlockSpec((B, H), lambda t: (0, 0)),
        ),
        out_shape=out_shapes,
        scratch_shapes=[pltpu.VMEM((B, H), jnp.float32)],
        compiler_params=pltpu.CompilerParams(
            dimension_semantics=("arbitrary",),
        ),
        interpret=interpret,
    )(emb2, e01, lens2, w2, whh_t, bias)


def kernel(src, lens, table, W_ih, W_hh, b_ih, b_hh):
    B, L = src.shape
    V, E = table.shape
    H = W_hh.shape[0]
    NW = 32

    ct = 1024
    half = -(-V // (2 * ct)) * ct
    n_blocks = half // ct
    table2 = _tc_table(table.T, n_blocks, ct)
    srcT = src.T.reshape(-1).astype(jnp.int32)
    idx_half = jnp.where(srcT < half, srcT, srcT - half)
    n_chunks = (L * B) // (NW * _CHUNK)
    idx3 = idx_half.reshape(NW, n_chunks, _CHUNK)
    rows = _sc_gather(table2, idx3)
    emb2 = rows.reshape(L, B, 2 * E)

    lens2 = lens.astype(jnp.int32).reshape(B, 1)
    wih_t = W_ih.T
    w2 = jnp.zeros((2 * E, 2 * H), jnp.float32)
    w2 = w2.at[:E, :H].set(wih_t).at[E:, H:].set(wih_t)
    bias = (b_ih + b_hh).reshape(1, H)

    e0m = ((src != 0) & (src < half)).astype(jnp.float32)
    e1m = (src >= half).astype(jnp.float32)
    e01 = jnp.concatenate([e0m, e1m], axis=1)

    out, hT = _tc_rnn(emb2, e01, lens2, w2, W_hh.T, bias)
    return jnp.transpose(out, (1, 0, 2)), hT[None]


# speedup vs baseline: 1.7121x; 1.0900x over previous
import functools

import jax
import jax.numpy as jnp
from jax import lax
from jax.experimental import pallas as pl
from jax.experimental.pallas import tpu as pltpu
from jax.experimental.pallas import tpu_sc as plsc

_CHUNK = 80


def _sc_gather(table2, idx3):
    V2, D = table2.shape
    NW, n_chunks, C = idx3.shape
    N = NW * n_chunks * C
    n_per_w = n_chunks * C
    half_chunks = n_chunks // 2
    half_rows = n_per_w // 2

    mesh = plsc.VectorSubcoreMesh(core_axis_name="c", subcore_axis_name="s")

    @functools.partial(
        pl.kernel,
        mesh=mesh,
        out_type=jax.ShapeDtypeStruct((N, D), jnp.float32),
        scratch_types=[
            pltpu.VMEM((n_chunks, C), jnp.int32),
            pltpu.VMEM((half_rows, D), jnp.float32),
            pltpu.SemaphoreType.DMA,
        ],
    )
    def gather_kernel(table_hbm, idx_hbm, out_hbm, idx_v, rows_v, sem):
        nc = lax.axis_index("c")
        ns = lax.axis_index("s")
        wid = ns * 2 + nc
        base = wid * n_per_w
        pltpu.sync_copy(idx_hbm.at[wid], idx_v)
        for half in range(2):
            copies = [
                pltpu.async_copy(
                    table_hbm.at[idx_v.at[half * half_chunks + j]],
                    rows_v.at[pl.ds(j * C, C)],
                    sem,
                )
                for j in range(half_chunks)
            ]
            for c in copies:
                c.wait()
            pltpu.sync_copy(
                rows_v, out_hbm.at[pl.ds(base + half * half_rows, half_rows)]
            )

    return gather_kernel(table2, idx3)


def _transpose_step(lo_ref, hi_ref, out_ref, *, E):
    out_ref[:, :E] = jnp.transpose(lo_ref[...], (1, 0))
    out_ref[:, E:] = jnp.transpose(hi_ref[...], (1, 0))


def _tc_table(tableT, n_blocks, ct):
    E, V = tableT.shape
    half = n_blocks * ct
    return pl.pallas_call(
        functools.partial(_transpose_step, E=E),
        grid=(n_blocks,),
        in_specs=[
            pl.BlockSpec((E, ct), lambda g: (0, g)),
            pl.BlockSpec(
                (E, ct),
                lambda g, nb=n_blocks, mx=(V - 1) // ct: (
                    0, jnp.minimum(g + nb, mx)),
            ),
        ],
        out_specs=pl.BlockSpec((ct, 2 * E), lambda g: (g, 0)),
        out_shape=jax.ShapeDtypeStruct((half, 2 * E), jnp.float32),
    )(tableT, tableT)


def _rnn_step(emb_ref, e01_ref, lens_ref, w2_ref, whh_ref, b_ref,
              out_ref, hid_ref, h_ref, *, L, H):
    t = pl.program_id(0)

    @pl.when(t == 0)
    def _init():
        h_ref[...] = jnp.zeros_like(h_ref)

    p = jnp.dot(emb_ref[0], w2_ref[...], preferred_element_type=jnp.float32)
    r = lax.broadcasted_iota(jnp.int32, (2 * L, 2), 0)
    c = lax.broadcasted_iota(jnp.int32, (2 * L, 2), 1)
    sel = (r == t + c * L).astype(jnp.float32)
    e = jnp.dot(e01_ref[...], sel, preferred_element_type=jnp.float32)
    e0 = e[:, 0:1]
    e1 = e[:, 1:2]
    x = p[:, :H] * e0 + p[:, H:] * e1
    h = h_ref[...]
    acc = x + jnp.dot(h, whh_ref[...], preferred_element_type=jnp.float32)
    h_new = jnp.tanh(acc + b_ref[...])
    valid = t < lens_ref[...]
    h_next = jnp.where(valid, h_new, h)
    h_ref[...] = h_next
    out_ref[0] = jnp.where(valid, h_new, 0.0)

    @pl.when(t == L - 1)
    def _fin():
        hid_ref[...] = h_next


def _tc_rnn(emb2, e01, lens2, w2, whh_t, bias, *, interpret=False):
    L, B, D = emb2.shape
    H = whh_t.shape[0]
    grid = (L,)
    out_shapes = (
        jax.ShapeDtypeStruct((L, B, H), jnp.float32),
        jax.ShapeDtypeStruct((B, H), jnp.float32),
    )
    return pl.pallas_call(
        functools.partial(_rnn_step, L=L, H=H),
        grid=grid,
        in_specs=[
            pl.BlockSpec((1, B, D), lambda t: (t, 0, 0)),
            pl.BlockSpec((B, 2 * L), lambda t: (0, 0)),
            pl.BlockSpec((B, 1), lambda t: (0, 0)),
            pl.BlockSpec((D, 2 * H), lambda t: (0, 0)),
            pl.BlockSpec((H, H), lambda t: (0, 0)),
            pl.BlockSpec((1, H), lambda t: (0, 0)),
        ],
        out_specs=(
            pl.BlockSpec((1, B, H), lambda t: (t, 0, 0)),
            pl.BlockSpec((B, H), lambda t: (0, 0)),
        ),
        out_shape=out_shapes,
        scratch_shapes=[pltpu.VMEM((B, H), jnp.float32)],
        compiler_params=pltpu.CompilerParams(
            dimension_semantics=("arbitrary",),
        ),
        interpret=interpret,
    )(emb2, e01, lens2, w2, whh_t, bias)


def kernel(src, lens, table, W_ih, W_hh, b_ih, b_hh):
    B, L = src.shape
    V, E = table.shape
    H = W_hh.shape[0]
    NW = 32

    ct = 2048
    half = -(-V // (2 * ct)) * ct
    n_blocks = half // ct
    table2 = _tc_table(table.T, n_blocks, ct)
    srcT = src.T.reshape(-1).astype(jnp.int32)
    idx_half = jnp.where(srcT < half, srcT, srcT - half)
    n_chunks = (L * B) // (NW * _CHUNK)
    idx3 = idx_half.reshape(NW, n_chunks, _CHUNK)
    rows = _sc_gather(table2, idx3)
    emb2 = rows.reshape(L, B, 2 * E)

    lens2 = lens.astype(jnp.int32).reshape(B, 1)
    wih_t = W_ih.T
    w2 = jnp.zeros((2 * E, 2 * H), jnp.float32)
    w2 = w2.at[:E, :H].set(wih_t).at[E:, H:].set(wih_t)
    bias = (b_ih + b_hh).reshape(1, H)

    e0m = ((src != 0) & (src < half)).astype(jnp.float32)
    e1m = (src >= half).astype(jnp.float32)
    e01 = jnp.concatenate([e0m, e1m], axis=1)

    out, hT = _tc_rnn(emb2, e01, lens2, w2, W_hh.T, bias)
    return jnp.transpose(out, (1, 0, 2)), hT[None]
